# Initial kernel scaffold; baseline (speedup 1.0000x reference)
import functools

import jax
import jax.numpy as jnp
from jax import lax
from jax.experimental import pallas as pl
from jax.experimental.pallas import tpu as pltpu
from jax.experimental.pallas import tpu_sc as plsc

D_MODEL = 64
B_TOTAL = 16384 * 50
NUM_WORKERS = 32
PER_WORKER = B_TOTAL // NUM_WORKERS
CHUNK = 128
NCHUNK = PER_WORKER // CHUNK
NGRID = B_TOTAL // CHUNK


def _make_kernel():
    mesh = plsc.VectorSubcoreMesh(core_axis_name="c", subcore_axis_name="s")

    @functools.partial(
        pl.kernel,
        mesh=mesh,
        out_type=jax.ShapeDtypeStruct((NGRID, CHUNK, D_MODEL), jnp.float32),
        scratch_types=[
            pltpu.VMEM((NCHUNK, CHUNK), jnp.int32),
            pltpu.VMEM((CHUNK, D_MODEL), jnp.float32),
            pltpu.SemaphoreType.DMA,
        ],
    )
    def k(table_hbm, idx_hbm, out_hbm, idx_v, rows_v, sem):
        num_cores = 2
        wid = lax.axis_index("s") * num_cores + lax.axis_index("c")
        pltpu.sync_copy(idx_hbm.at[wid], idx_v)

        def body(j, carry):
            pltpu.async_copy(table_hbm.at[idx_v.at[j]], rows_v, sem).wait()
            pltpu.sync_copy(rows_v, out_hbm.at[wid * NCHUNK + j])
            return carry

        lax.fori_loop(0, NCHUNK, body, 0)

    return k


_gather_kernel = _make_kernel()


@jax.jit
def kernel(x, table):
    idx = x.reshape(NUM_WORKERS, NCHUNK, CHUNK)
    out = _gather_kernel(table, idx)
    return out.reshape(x.shape[0], x.shape[1], D_MODEL)


# speedup vs baseline: 5.2178x; 5.2178x over previous
import functools

import jax
import jax.numpy as jnp
from jax import lax
from jax.experimental import pallas as pl
from jax.experimental.pallas import tpu as pltpu
from jax.experimental.pallas import tpu_sc as plsc

D_MODEL = 64
B_TOTAL = 16384 * 50
NUM_WORKERS = 32
PER_WORKER = B_TOTAL // NUM_WORKERS
CHUNK = 128
NCHUNK = PER_WORKER // CHUNK
NGRID = B_TOTAL // CHUNK


def _make_kernel():
    mesh = plsc.VectorSubcoreMesh(core_axis_name="c", subcore_axis_name="s")

    @functools.partial(
        pl.kernel,
        mesh=mesh,
        out_type=jax.ShapeDtypeStruct((NGRID, CHUNK, D_MODEL), jnp.float32),
        scratch_types=[
            pltpu.VMEM((NCHUNK, CHUNK), jnp.int32),
            pltpu.VMEM((CHUNK, D_MODEL), jnp.float32),
            pltpu.SemaphoreType.DMA,
        ],
        compiler_params=pltpu.CompilerParams(use_tc_tiling_on_sc=False),
    )
    def k(table_hbm, idx_hbm, out_hbm, idx_v, rows_v, sem):
        num_cores = 2
        wid = lax.axis_index("s") * num_cores + lax.axis_index("c")
        pltpu.sync_copy(idx_hbm.at[wid], idx_v)

        def body(j, carry):
            pltpu.async_copy(table_hbm.at[idx_v.at[j]], rows_v, sem).wait()
            pltpu.sync_copy(rows_v, out_hbm.at[wid * NCHUNK + j])
            return carry

        lax.fori_loop(0, NCHUNK, body, 0)

    return k


_gather_kernel = _make_kernel()


@jax.jit
def kernel(x, table):
    idx = x.reshape(NUM_WORKERS, NCHUNK, CHUNK)
    out = _gather_kernel(table, idx)
    return out.reshape(x.shape[0], x.shape[1], D_MODEL)


# speedup vs baseline: 6.2359x; 1.1951x over previous
import functools

import jax
import jax.numpy as jnp
from jax import lax
from jax.experimental import pallas as pl
from jax.experimental.pallas import tpu as pltpu
from jax.experimental.pallas import tpu_sc as plsc

D_MODEL = 64
B_TOTAL = 16384 * 50
NUM_WORKERS = 32
PER_WORKER = B_TOTAL // NUM_WORKERS
CHUNK = 128
NCHUNK = PER_WORKER // CHUNK
NGRID = B_TOTAL // CHUNK
NBUF = 8
NGROUPS = NCHUNK // NBUF


def _make_kernel():
    mesh = plsc.VectorSubcoreMesh(core_axis_name="c", subcore_axis_name="s")

    @functools.partial(
        pl.kernel,
        mesh=mesh,
        out_type=jax.ShapeDtypeStruct((NGRID, CHUNK, D_MODEL), jnp.float32),
        scratch_types=[
            pltpu.VMEM((NCHUNK, CHUNK), jnp.int32),
            pltpu.VMEM((NBUF, CHUNK, D_MODEL), jnp.float32),
            pltpu.SemaphoreType.DMA,
            pltpu.SemaphoreType.DMA,
        ],
        compiler_params=pltpu.CompilerParams(use_tc_tiling_on_sc=False),
    )
    def k(table_hbm, idx_hbm, out_hbm, idx_v, bufs, gsem, wsem):
        num_cores = 2
        wid = lax.axis_index("s") * num_cores + lax.axis_index("c")
        pltpu.sync_copy(idx_hbm.at[wid], idx_v)
        out_base = wid * NCHUNK

        def gather(j, b):
            return pltpu.make_async_copy(
                table_hbm.at[idx_v.at[j]], bufs.at[b], gsem)

        def wback(j, b):
            return pltpu.make_async_copy(
                bufs.at[b], out_hbm.at[out_base + j], wsem)

        for b in range(NBUF):
            gather(b, b).start()

        def group(g, carry):
            g0 = g * NBUF
            for b in range(NBUF):
                gather(g0 + b, b).wait()
                wback(g0 + b, b).start()
            for b in range(NBUF):
                wback(g0 + b, b).wait()
                gather(g0 + NBUF + b, b).start()
            return carry

        lax.fori_loop(0, NGROUPS - 1, group, 0)

        g0 = (NGROUPS - 1) * NBUF
        for b in range(NBUF):
            gather(g0 + b, b).wait()
            wback(g0 + b, b).start()
        for b in range(NBUF):
            wback(g0 + b, b).wait()

    return k


_gather_kernel = _make_kernel()


@jax.jit
def kernel(x, table):
    idx = x.reshape(NUM_WORKERS, NCHUNK, CHUNK)
    out = _gather_kernel(table, idx)
    return out.reshape(x.shape[0], x.shape[1], D_MODEL)
